# baseline (speedup 1.0000x reference)
import jax
import jax.numpy as jnp
from jax import lax
from jax.experimental import pallas as pl
from jax.experimental.pallas import tpu as pltpu
from jax.experimental.pallas import tpu_sc as plsc

N = 10000
E = 320000
IN_CH = 128
OUT_CH = 128
H = 8
B = 4
A = 3
DH = 16
F = B * DH
FD = 128
CW = H * B * A

NW = 32
NB = 320
NPAD = NW * NB
CH = 4000
NCHUNK = E // CH
G = 128
RB = 512
NBLK = NPAD // RB


def _tc1_body(x_ref, wb_ref, cw_ref, cb_ref, flat_ref, logit_ref):
    xb = x_ref[...]
    mm = jnp.dot(xb, wb_ref[...], preferred_element_type=jnp.float32)
    flat_ref[...] = jnp.concatenate(
        [mm, jnp.zeros((RB, FD - F), jnp.float32)], axis=1
    )
    logit_ref[...] = (
        jnp.dot(xb, cw_ref[...], preferred_element_type=jnp.float32) + cb_ref[...]
    )


_tc1 = pl.pallas_call(
    _tc1_body,
    grid=(NBLK,),
    in_specs=[
        pl.BlockSpec((RB, IN_CH), lambda i: (i, 0)),
        pl.BlockSpec((IN_CH, F), lambda i: (0, 0)),
        pl.BlockSpec((IN_CH, CW), lambda i: (0, 0)),
        pl.BlockSpec((1, CW), lambda i: (0, 0)),
    ],
    out_specs=[
        pl.BlockSpec((RB, FD), lambda i: (i, 0)),
        pl.BlockSpec((RB, CW), lambda i: (i, 0)),
    ],
    out_shape=[
        jax.ShapeDtypeStruct((NPAD, FD), jnp.float32),
        jax.ShapeDtypeStruct((NPAD, CW), jnp.float32),
    ],
)


def _sc_body(flat_hbm, src_hbm, dst_hbm, add_out, max_out, cnt_out,
             src_v, dst_v, wl_d, wl_s, rows_v, acc_add, acc_max, acc_cnt, sem):
    cid = lax.axis_index("c")
    sid = lax.axis_index("s")
    wid = sid * 2 + cid
    n0 = wid * NB

    zf = jnp.zeros((16,), jnp.float32)
    ninf = jnp.full((16,), -jnp.inf, jnp.float32)
    zi = jnp.zeros((16,), jnp.int32)
    ones_f = jnp.ones((16,), jnp.float32)

    def zero_acc(i, c):
        acc_add[pl.ds(i * 16, 16)] = zf
        acc_max[pl.ds(i * 16, 16)] = ninf
        return c
    lax.fori_loop(0, (NB + 1) * F // 16, zero_acc, 0)

    def zero_cnt(i, c):
        acc_cnt[pl.ds(i * 16, 16)] = zf
        return c
    lax.fori_loop(0, NB + 1, zero_cnt, 0)

    def zero_wl(i, c):
        wl_s[pl.ds(i * 16, 16)] = zi
        return c
    lax.fori_loop(0, (CH + 16) // 16, zero_wl, 0)

    def chunk_body(co, carry):
        pltpu.sync_copy(src_hbm.at[pl.ds(co * CH, CH)], src_v)
        pltpu.sync_copy(dst_hbm.at[pl.ds(co * CH, CH)], dst_v)

        lane = lax.broadcasted_iota(jnp.int32, (16,), 0)

        def filt(i, p):
            d = dst_v[pl.ds(i * 16, 16)]
            sv = src_v[pl.ds(i * 16, 16)]
            dl = d - n0
            mi = 1 + ((dl | (NB - 1 - dl)) >> 31)
            ms = [mi[l] for l in range(16)]
            h = ms[0]
            for l in range(1, 16):
                h = h + ms[l]

            @pl.when(h > 0)
            def _():
                dl_out = jnp.zeros((16,), jnp.int32)
                sv_out = jnp.zeros((16,), jnp.int32)
                c = ms[0] * 0
                for l in range(16):
                    t = lane - c
                    ind = (1 - jnp.minimum(t * t, 1)) * ms[l]
                    dl_out = dl_out + ind * dl[l]
                    sv_out = sv_out + ind * sv[l]
                    c = c + ms[l]
                ge = 1 + ((lane - c) >> 31)
                dl_out = dl_out + ge * NB
                wl_d[pl.ds(p, 16)] = dl_out
                wl_s[pl.ds(p, 16)] = sv_out

            return p + h

        p = lax.fori_loop(0, CH // 16, filt, 0)

        wl_d[pl.ds(p, 16)] = jnp.full((16,), NB, jnp.int32)
        wl_s[pl.ds(p, 16)] = zi

        nv = (p + 15) // 16
        ng = (p + (G - 1)) // G
        GV = G // 16

        def gbody(g, c):
            pltpu.async_copy(flat_hbm.at[wl_s.at[pl.ds(g * G, G)]], rows_v, sem).wait()
            nvg = jnp.minimum(nv - g * GV, GV)

            def vbody(vi, c2):
                dl16 = wl_d[pl.ds(g * G + vi * 16, 16)]
                off16 = dl16 * F
                cof16 = dl16 * 16
                for l in range(16):
                    b = off16[l]
                    el = vi * 16 + l
                    for j in range(4):
                        r = rows_v[el, pl.ds(j * 16, 16)]
                        o = pl.ds(b + j * 16, 16)
                        acc_add[o] = acc_add[o] + r
                        acc_max[o] = jnp.maximum(acc_max[o], r)
                    oc = pl.ds(cof16[l], 16)
                    acc_cnt[oc] = acc_cnt[oc] + ones_f
                return c2

            lax.fori_loop(0, nvg, vbody, 0)
            return c

        lax.fori_loop(0, ng, gbody, 0)
        return carry

    lax.fori_loop(0, NCHUNK, chunk_body, 0)

    pltpu.sync_copy(acc_add.at[pl.ds(0, NB * F)], add_out.at[pl.ds(n0 * F, NB * F)])
    pltpu.sync_copy(acc_max.at[pl.ds(0, NB * F)], max_out.at[pl.ds(n0 * F, NB * F)])
    pltpu.sync_copy(acc_cnt.at[pl.ds(0, NB * 16)], cnt_out.at[pl.ds(n0 * 16, NB * 16)])


_sc_seg = pl.kernel(
    _sc_body,
    mesh=plsc.VectorSubcoreMesh(core_axis_name="c", subcore_axis_name="s"),
    out_type=(
        jax.ShapeDtypeStruct((NPAD * F,), jnp.float32),
        jax.ShapeDtypeStruct((NPAD * F,), jnp.float32),
        jax.ShapeDtypeStruct((NPAD * 16,), jnp.float32),
    ),
    scratch_types=[
        pltpu.VMEM((CH,), jnp.int32),
        pltpu.VMEM((CH,), jnp.int32),
        pltpu.VMEM((CH + 16,), jnp.int32),
        pltpu.VMEM((CH + 16,), jnp.int32),
        pltpu.VMEM((G, FD), jnp.float32),
        pltpu.VMEM(((NB + 1) * F,), jnp.float32),
        pltpu.VMEM(((NB + 1) * F,), jnp.float32),
        pltpu.VMEM(((NB + 1) * 16,), jnp.float32),
        pltpu.SemaphoreType.DMA,
    ],
)


def _tc2_body(add_ref, max_ref, cnt_ref, logit_ref, bias_ref, out_ref):
    add = add_ref[...]
    cnt = cnt_ref[...][:, 0:1]
    mx = jnp.where(cnt > 0.0, max_ref[...], 0.0)
    mean = add / jnp.maximum(cnt, 1.0)
    lg = logit_ref[...]
    aggs = (add, mean, mx)
    pieces = []
    for h in range(H):
        sl = lg[:, h * B * A:(h + 1) * B * A]
        m = jnp.max(sl, axis=1, keepdims=True)
        e = jnp.exp(sl - m)
        w = e / jnp.sum(e, axis=1, keepdims=True)
        zh = jnp.zeros((RB, DH), jnp.float32)
        for b in range(B):
            for a in range(A):
                c = b * A + a
                zh = zh + w[:, c:c + 1] * aggs[a][:, b * DH:(b + 1) * DH]
        pieces.append(zh)
    out_ref[...] = jnp.concatenate(pieces, axis=1) + bias_ref[...]


_tc2 = pl.pallas_call(
    _tc2_body,
    grid=(NBLK,),
    in_specs=[
        pl.BlockSpec((RB, F), lambda i: (i, 0)),
        pl.BlockSpec((RB, F), lambda i: (i, 0)),
        pl.BlockSpec((RB, 16), lambda i: (i, 0)),
        pl.BlockSpec((RB, CW), lambda i: (i, 0)),
        pl.BlockSpec((1, OUT_CH), lambda i: (0, 0)),
    ],
    out_specs=pl.BlockSpec((RB, OUT_CH), lambda i: (i, 0)),
    out_shape=jax.ShapeDtypeStruct((NPAD, OUT_CH), jnp.float32),
)


def kernel(x, edge_index, W_bases, comb_W, comb_b, bias):
    x = x.astype(jnp.float32)
    src = edge_index[0].astype(jnp.int32)
    dst = edge_index[1].astype(jnp.int32)
    wb2 = jnp.transpose(W_bases, (1, 0, 2)).reshape(IN_CH, F)
    cwT = comb_W.T
    cb2 = comb_b.reshape(1, CW)
    bias2 = bias.reshape(1, OUT_CH)
    x_pad = jnp.pad(x, ((0, NPAD - N), (0, 0)))

    flat, logits = _tc1(x_pad, wb2, cwT, cb2)
    add_f, max_f, cnt_f = _sc_seg(flat, src, dst)
    z = _tc2(
        add_f.reshape(NPAD, F),
        max_f.reshape(NPAD, F),
        cnt_f.reshape(NPAD, 16),
        logits,
        bias2,
    )
    return z[:N]
